# baseline (speedup 1.0000x reference)
import jax
import jax.numpy as jnp
from jax import lax
from jax.experimental import pallas as pl
from jax.experimental.pallas import tpu as pltpu

CH = 3
FEAT = 16
IMG = 64
SLOTS = 1000
READ = 3
BATCH = 8


def _conv2d(x, w, stride, pad):
    return jax.lax.conv_general_dilated(
        x, w, (stride, stride), [(pad, pad), (pad, pad)],
        dimension_numbers=('NCHW', 'OIHW', 'NCHW'))


def _batchnorm(x, eps=1e-5):
    mean = x.mean(axis=(0, 2, 3), keepdims=True)
    var = x.var(axis=(0, 2, 3), keepdims=True)
    return (x - mean) / jnp.sqrt(var + eps)


def _fast_att(x, w1, b1, w2, b2):
    y = jax.nn.softmax(x @ w1.T + b1, axis=1)
    y = y @ w2.T + b2
    return x * y


def _encode(inputs, p):
    B = inputs.shape[0]
    h = jax.nn.relu(_batchnorm(_conv2d(inputs, p['enc1'], 4, 1)))
    h = jax.nn.relu(_batchnorm(_conv2d(h, p['enc2'], 2, 1)))
    h = jax.nn.relu(_batchnorm(_conv2d(h, p['enc3'], 2, 1)))
    h = jax.nn.relu(_batchnorm(_conv2d(h, p['enc4'], 1, 0)))
    return h.reshape(B, -1)


def _fast_att_k(x, w1, b1, w2, b2):
    y = lax.dot_general(x, w1, (((1,), (1,)), ((), ()))) + b1[None, :]
    mx = jnp.max(y, axis=1, keepdims=True)
    e = jnp.exp(y - mx)
    y = e / jnp.sum(e, axis=1, keepdims=True)
    y = lax.dot_general(y, w2, (((1,), (1,)), ((), ()))) + b2[None, :]
    return x * y


def _heads_body(enc_ref, rw1_ref, rb1_ref, rw2_ref, rb2_ref, rw_ref, rb_ref,
                uw1_ref, ub1_ref, uw2_ref, ub2_ref, uwh_ref, ubh_ref,
                idx_out, readw_out, uw_out):
    enc = enc_ref[...]
    rfa = _fast_att_k(enc, rw1_ref[...], rb1_ref[...],
                      rw2_ref[...], rb2_ref[...])
    read = jnp.tanh(
        lax.dot_general(rfa, rw_ref[...], (((1,), (1,)), ((), ())))
        + rb_ref[...][None, :])
    idx_cols, rw_cols = [], []
    for s in range(READ):
        chunk = read[:, s * SLOTS:(s + 1) * SLOTS]
        mx = jnp.max(chunk, axis=1, keepdims=True)
        iota = lax.broadcasted_iota(jnp.int32, (BATCH, SLOTS), 1)
        first = jnp.min(jnp.where(chunk == mx, iota, SLOTS), axis=1,
                        keepdims=True)
        idx_cols.append(first)
        rw_cols.append(mx)
    idx_out[...] = jnp.concatenate(idx_cols, axis=1)
    readw_out[...] = jnp.concatenate(rw_cols, axis=1)
    ufa = _fast_att_k(enc, uw1_ref[...], ub1_ref[...],
                      uw2_ref[...], ub2_ref[...])
    u = lax.dot_general(ufa, uwh_ref[...], (((1,), (1,)), ((), ())))
    uw_out[...] = jax.nn.sigmoid(u + ubh_ref[...][None, :])


def _heads_pallas(encoded, p, interpret=False):
    return pl.pallas_call(
        _heads_body,
        out_shape=[
            jax.ShapeDtypeStruct((BATCH, READ), jnp.int32),
            jax.ShapeDtypeStruct((BATCH, READ), jnp.float32),
            jax.ShapeDtypeStruct((BATCH, READ), jnp.float32),
        ],
        interpret=interpret,
    )(encoded, p['rfa_w1'], p['rfa_b1'], p['rfa_w2'], p['rfa_b2'],
      p['r_w'], p['r_b'],
      p['ufa_w1'], p['ufa_b1'], p['ufa_w2'], p['ufa_b2'],
      p['u_w'], p['u_b'])


def _shifts9(x):
    z1 = jnp.zeros_like(x[..., :1])
    zrow = jnp.zeros_like(x[..., :1, :])
    out = {}
    for dy in (-1, 0, 1):
        if dy == 0:
            base = x
        elif dy == 1:
            up = jnp.concatenate([x[..., 1:, :], zrow], axis=-2)
            base = jnp.concatenate([x[..., 64:], up[..., :64]], axis=-1)
        else:
            dn = jnp.concatenate([zrow, x[..., :-1, :]], axis=-2)
            base = jnp.concatenate([dn[..., 64:], x[..., :64]], axis=-1)
        for dx in (-1, 0, 1):
            if dx == 0:
                t = base
            elif dx == 1:
                t = jnp.concatenate(
                    [base[..., 1:64], z1, base[..., 65:128], z1], axis=-1)
            else:
                t = jnp.concatenate(
                    [z1, base[..., 0:63], z1, base[..., 64:127]], axis=-1)
            out[(dy, dx)] = t
    return out


def _conv3x3(x, w):
    co, ci = w.shape[0], w.shape[1]
    sh = _shifts9(x)
    outs = []
    for o in range(co):
        acc = None
        for i in range(ci):
            for ky in range(3):
                for kx in range(3):
                    t = w[o, i, ky, kx] * sh[(ky - 1, kx - 1)][:, i]
                    acc = t if acc is None else acc + t
        outs.append(acc[:, None])
    return jnp.concatenate(outs, axis=1)


def _softmax_h(x):
    m1 = jnp.max(x, axis=-2, keepdims=True)
    m64 = jnp.maximum(m1[..., :64], m1[..., 64:])
    e = jnp.exp(x - jnp.concatenate([m64, m64], axis=-1))
    s1 = jnp.sum(e, axis=-2, keepdims=True)
    s64 = s1[..., :64] + s1[..., 64:]
    return e / jnp.concatenate([s64, s64], axis=-1)


def _fast_att_img_k(x, w1, w2):
    y = _conv3x3(x, w1)
    y = _softmax_h(y)
    y = _conv3x3(y, w2)
    return x * y


def _slot_copies(mem_ref, idx_ref, rbuf_ref, sem_ref, g, slot):
    return [
        pltpu.make_async_copy(
            mem_ref.at[b, idx_ref[g * BATCH + b]],
            rbuf_ref.at[slot, b],
            sem_ref.at[slot],
        )
        for b in range(BATCH)
    ]


def _update_body(idx_ref, mem_ref, inp_ref, uw_ref, rw_ref,
                 um1_ref, um2_ref, um3_ref, am1_ref, am2_ref, am3_ref,
                 out_ref, m_ref, rbuf_ref, sem_ref):
    s = pl.program_id(0)

    @pl.when(s == 0)
    def _():
        m_ref[...] = jnp.zeros((BATCH, CH, IMG // 2, 2 * IMG), jnp.float32)
        for c in _slot_copies(mem_ref, idx_ref, rbuf_ref, sem_ref, 0, 0):
            c.start()

    @pl.when(s < READ - 1)
    def _():
        for c in _slot_copies(mem_ref, idx_ref, rbuf_ref, sem_ref,
                              s + 1, (s + 1) % 2):
            c.start()

    for c in _slot_copies(mem_ref, idx_ref, rbuf_ref, sem_ref, s, s % 2):
        c.wait()

    r = rbuf_ref[s % 2]
    inp = inp_ref[...]
    um1, um2, um3 = um1_ref, um2_ref, um3_ref
    am1, am2, am3 = am1_ref, am2_ref, am3_ref

    x = jnp.concatenate([r, inp], axis=1)
    um = _fast_att_img_k(x, um1, um2)
    um = jax.nn.relu(_conv3x3(um, um3))
    w = uw_ref[...].reshape(BATCH, 1, 1, 1)
    r2 = w * um + (1.0 - w) * r

    x2 = jnp.concatenate([r2, m_ref[...]], axis=1)
    am = _fast_att_img_k(x2, am1, am2)
    am = jax.nn.relu(_conv3x3(am, am3))
    m = rw_ref[...].reshape(BATCH, 1, 1, 1) * am
    m_ref[...] = m
    out_ref[...] = jnp.tanh(m)


def _update_pallas(memory, idx_flat, inputs, uw, read_w, p, interpret=False):
    uw3 = jnp.transpose(uw).reshape(READ, BATCH, 1)
    rw3 = jnp.transpose(read_w).reshape(READ, BATCH, 1)
    full4 = pl.BlockSpec((BATCH, CH, IMG // 2, 2 * IMG),
                         lambda s, *_: (0, 0, 0, 0))
    scal = pl.BlockSpec((1, BATCH, 1), lambda s, *_: (s, 0, 0))
    w66 = pl.BlockSpec(memory_space=pltpu.SMEM)
    w36 = pl.BlockSpec(memory_space=pltpu.SMEM)
    grid_spec = pltpu.PrefetchScalarGridSpec(
        num_scalar_prefetch=1,
        grid=(READ,),
        in_specs=[pl.BlockSpec(memory_space=pl.ANY),
                  full4, scal, scal, w66, w66, w36, w66, w66, w36],
        out_specs=full4,
        scratch_shapes=[
            pltpu.VMEM((BATCH, CH, IMG // 2, 2 * IMG), jnp.float32),
            pltpu.VMEM((2, BATCH, CH, IMG // 2, 2 * IMG), jnp.float32),
            pltpu.SemaphoreType.DMA((2,)),
        ],
    )
    return pl.pallas_call(
        _update_body,
        grid_spec=grid_spec,
        out_shape=jax.ShapeDtypeStruct((BATCH, CH, IMG // 2, 2 * IMG),
                                       jnp.float32),
        interpret=interpret,
    )(idx_flat, memory, inputs, uw3, rw3,
      p['um1'], p['um2'], p['um3'], p['am1'], p['am2'], p['am3'])


def kernel(inputs, memory, params):
    encoded = _encode(inputs, params)
    read_idx, read_w, uw = _heads_pallas(encoded, params)
    idx_flat = jnp.transpose(read_idx).reshape(READ * BATCH)
    mem_v = memory.reshape(BATCH, SLOTS, CH, IMG // 2, 2 * IMG)
    inp_v = inputs.reshape(BATCH, CH, IMG // 2, 2 * IMG)
    out = _update_pallas(mem_v, idx_flat, inp_v, uw, read_w, params)
    return out.reshape(BATCH, CH, IMG, IMG)
